# baseline (speedup 1.0000x reference)
import jax
import jax.numpy as jnp
from jax.experimental import pallas as pl


def _mean3_body(x0_ref, x1_ref, x2_ref, o_ref):
    o_ref[...] = (x0_ref[...] + x1_ref[...] + x2_ref[...]) * (1.0 / 3.0)


def kernel(edge_index, xs_0, xs_1, xs_2):
    n, d = xs_0.shape
    nn, dd = n // 2, d * 2
    block = 2000
    while nn % block != 0:
        block //= 2
    spec = pl.BlockSpec((block, dd), lambda i: (i, 0))
    out = pl.pallas_call(
        _mean3_body,
        grid=(nn // block,),
        in_specs=[spec, spec, spec],
        out_specs=spec,
        out_shape=jax.ShapeDtypeStruct((nn, dd), xs_0.dtype),
    )(xs_0.reshape(nn, dd), xs_1.reshape(nn, dd), xs_2.reshape(nn, dd))
    return out.reshape(n, d)


# speedup vs baseline: 4.4451x; 4.4451x over previous
import jax
import jax.numpy as jnp
from jax.experimental import pallas as pl


def _mean3_body(x0_ref, x1_ref, x2_ref, o_ref):
    o_ref[...] = (x0_ref[...] + x1_ref[...] + x2_ref[...]) * (1.0 / 3.0)


def kernel(edge_index, xs_0, xs_1, xs_2):
    n, d = xs_0.shape
    block = 5000
    while n % block != 0:
        block //= 2
    spec = pl.BlockSpec((block, d), lambda i: (i, 0))
    return pl.pallas_call(
        _mean3_body,
        grid=(n // block,),
        in_specs=[spec, spec, spec],
        out_specs=spec,
        out_shape=jax.ShapeDtypeStruct((n, d), xs_0.dtype),
    )(xs_0, xs_1, xs_2)
